# baseline (speedup 1.0000x reference)
import functools

import jax
import jax.numpy as jnp
from jax import lax
from jax.experimental import pallas as pl
from jax.experimental.pallas import tpu as pltpu
from jax.experimental.pallas import tpu_sc as plsc

B, T, N, C, K = 8, 32, 256, 768, 10
R = B * K
ROW = N * C
NCH = 8
CHUNK = ROW // NCH
NW = 32
TPW = R * NCH // NW
RB = 4
APW = R // 16


def _body(idx_hbm, patch_hbm, audio_hbm, out_patch, out_audio,
          idx_v, bufs, abuf, rs0, rs1, rs2, rs3, ws0, ws1, ws2, ws3, asem):
    c = lax.axis_index("c")
    s = lax.axis_index("s")
    w = s * 2 + c

    pltpu.sync_copy(idx_hbm, idx_v)
    iota = lax.iota(jnp.int32, 16)

    def extract(r):
        base = pl.multiple_of((r // 16) * 16, 16)
        vec = idx_v[pl.ds(base, 16)]
        return jnp.max(jnp.where(iota == r % 16, vec, 0))

    @pl.when(w < 16)
    def _audio_in():
        for j in range(APW):
            r = w * APW + j
            src_row = (r // K) * T + extract(r)
            pltpu.async_copy(audio_hbm.at[pl.ds(src_row, 1)],
                             abuf.at[j], asem)

    rsem = [rs0, rs1, rs2, rs3]
    wsem = [ws0, ws1, ws2, ws3]
    rh = [None] * RB
    wh = [None] * RB

    def read(j):
        g = w * TPW + j
        r = g // NCH
        ch = g % NCH
        src_row = ((r // K) * T + extract(r)) * NCH + ch
        return pltpu.async_copy(patch_hbm.at[pl.ds(src_row, 1)],
                                bufs.at[j % RB], rsem[j % RB])

    def write(j):
        g = w * TPW + j
        return pltpu.async_copy(bufs.at[j % RB],
                                out_patch.at[pl.ds(g, 1)], wsem[j % RB])

    for j in range(TPW):
        if wh[j % RB] is not None:
            wh[j % RB].wait()
        rh[j % RB] = read(j)
        if j >= RB - 1:
            i = j - (RB - 1)
            rh[i % RB].wait()
            wh[i % RB] = write(i)
    for i in range(TPW - (RB - 1), TPW):
        rh[i % RB].wait()
        wh[i % RB] = write(i)
    for h in wh:
        h.wait()

    @pl.when(w < 16)
    def _audio_out():
        for j in range(APW):
            pltpu.make_async_copy(audio_hbm.at[pl.ds(0, 1)],
                                  abuf.at[j], asem).wait()
            pltpu.sync_copy(abuf.at[j],
                            out_audio.at[pl.ds(w * APW + j, 1)])


@jax.jit
def _gather_call(idx, patch2d, audio2d):
    mesh = plsc.VectorSubcoreMesh(core_axis_name="c", subcore_axis_name="s")
    run = functools.partial(
        pl.kernel,
        mesh=mesh,
        compiler_params=pltpu.CompilerParams(needs_layout_passes=False),
        out_type=(
            jax.ShapeDtypeStruct((R * NCH, CHUNK), jnp.float32),
            jax.ShapeDtypeStruct((R, C), jnp.float32),
        ),
        scratch_types=[
            pltpu.VMEM((R,), jnp.int32),
            pltpu.VMEM((RB, 1, CHUNK), jnp.float32),
            pltpu.VMEM((APW, 1, C), jnp.float32),
            pltpu.SemaphoreType.DMA,
            pltpu.SemaphoreType.DMA,
            pltpu.SemaphoreType.DMA,
            pltpu.SemaphoreType.DMA,
            pltpu.SemaphoreType.DMA,
            pltpu.SemaphoreType.DMA,
            pltpu.SemaphoreType.DMA,
            pltpu.SemaphoreType.DMA,
            pltpu.SemaphoreType.DMA,
        ],
    )(_body)
    return run(idx, patch2d, audio2d)


def kernel(top_k_index_sort, patch_feat, audio_feat):
    idx = top_k_index_sort.reshape(R).astype(jnp.int32)
    patch2d = patch_feat.reshape(B * T * NCH, CHUNK)
    audio2d = audio_feat.reshape(B * T, C)
    out_p, out_a = _gather_call(idx, patch2d, audio2d)
    return out_p.reshape(B, K, N, C), out_a.reshape(B, K, C)


# speedup vs baseline: 2.6206x; 2.6206x over previous
import functools

import jax
import jax.numpy as jnp
from jax.experimental import pallas as pl
from jax.experimental.pallas import tpu as pltpu

B, T, N, C, K = 8, 32, 256, 768, 10
R = B * K


def _body(idx_sm, patch_blk, audio_blk, out_patch_blk, out_audio_blk):
    out_patch_blk[...] = patch_blk[...]
    out_audio_blk[...] = audio_blk[...]


@jax.jit
def _gather_call(idx, patch, audio):
    grid_spec = pltpu.PrefetchScalarGridSpec(
        num_scalar_prefetch=1,
        grid=(R, 2),
        in_specs=[
            pl.BlockSpec((1, 1, N // 2, C),
                         lambda i, j, idx_ref: (i // K, idx_ref[i], j, 0)),
            pl.BlockSpec((1, 1, 1, C), lambda i, j, idx_ref: (i // K, idx_ref[i], 0, 0)),
        ],
        out_specs=[
            pl.BlockSpec((1, 1, N // 2, C),
                         lambda i, j, idx_ref: (i // K, i % K, j, 0)),
            pl.BlockSpec((1, 1, 1, C), lambda i, j, idx_ref: (i // K, i % K, 0, 0)),
        ],
    )
    return pl.pallas_call(
        _body,
        grid_spec=grid_spec,
        out_shape=(
            jax.ShapeDtypeStruct((B, K, N, C), jnp.float32),
            jax.ShapeDtypeStruct((B, K, 1, C), jnp.float32),
        ),
    )(idx, patch, audio)


def kernel(top_k_index_sort, patch_feat, audio_feat):
    idx = top_k_index_sort.reshape(R).astype(jnp.int32)
    out_p, out_a = _gather_call(idx, patch_feat, audio_feat.reshape(B, T, 1, C))
    return out_p, out_a.reshape(B, K, C)


# speedup vs baseline: 3.6699x; 1.4004x over previous
import functools

import jax
import jax.numpy as jnp
from jax.experimental import pallas as pl
from jax.experimental.pallas import tpu as pltpu

B, T, N, C, K = 8, 32, 256, 768, 10
R = B * K


def _body(t_sm, o_sm, patch_blk, audio_blk, out_patch_blk, out_audio_blk):
    out_patch_blk[...] = patch_blk[...]
    out_audio_blk[...] = audio_blk[...]


@jax.jit
def _gather_call(tsrt, korder, patch, audio):
    grid_spec = pltpu.PrefetchScalarGridSpec(
        num_scalar_prefetch=2,
        grid=(R,),
        in_specs=[
            pl.BlockSpec((1, 1, N, C),
                         lambda i, t_ref, o_ref: (i // K, t_ref[i], 0, 0)),
            pl.BlockSpec((1, 1, 1, C),
                         lambda i, t_ref, o_ref: (i // K, t_ref[i], 0, 0)),
        ],
        out_specs=[
            pl.BlockSpec((1, 1, N, C),
                         lambda i, t_ref, o_ref: (i // K, o_ref[i], 0, 0)),
            pl.BlockSpec((1, 1, 1, C),
                         lambda i, t_ref, o_ref: (i // K, o_ref[i], 0, 0)),
        ],
    )
    return pl.pallas_call(
        _body,
        grid_spec=grid_spec,
        out_shape=(
            jax.ShapeDtypeStruct((B, K, N, C), jnp.float32),
            jax.ShapeDtypeStruct((B, K, 1, C), jnp.float32),
        ),
    )(tsrt, korder, patch, audio)


def kernel(top_k_index_sort, patch_feat, audio_feat):
    idx2 = top_k_index_sort.reshape(B, K).astype(jnp.int32)
    korder = jnp.argsort(idx2, axis=1).astype(jnp.int32)
    tsrt = jnp.take_along_axis(idx2, korder, axis=1)
    out_p, out_a = _gather_call(tsrt.reshape(R), korder.reshape(R),
                                patch_feat, audio_feat.reshape(B, T, 1, C))
    return out_p, out_a.reshape(B, K, C)


# speedup vs baseline: 3.9294x; 1.0707x over previous
import functools

import jax
import jax.numpy as jnp
from jax.experimental import pallas as pl
from jax.experimental.pallas import tpu as pltpu

B, T, N, C, K = 8, 32, 256, 768, 10
R = B * K


def _body(idx_sm, patch_blk, audio_blk, out_patch_blk, out_audio_blk):
    out_patch_blk[...] = patch_blk[...]
    out_audio_blk[...] = audio_blk[...]


@jax.jit
def _gather_call(idx, patch, audio):
    grid_spec = pltpu.PrefetchScalarGridSpec(
        num_scalar_prefetch=1,
        grid=(R,),
        in_specs=[
            pl.BlockSpec((1, 1, N, C),
                         lambda i, idx_ref: (i // K, idx_ref[i // K, 0, i % K], 0, 0)),
            pl.BlockSpec((1, 1, 1, C),
                         lambda i, idx_ref: (i // K, idx_ref[i // K, 0, i % K], 0, 0)),
        ],
        out_specs=[
            pl.BlockSpec((1, 1, N, C), lambda i, idx_ref: (i // K, i % K, 0, 0)),
            pl.BlockSpec((1, 1, 1, C), lambda i, idx_ref: (i // K, i % K, 0, 0)),
        ],
    )
    return pl.pallas_call(
        _body,
        grid_spec=grid_spec,
        out_shape=(
            jax.ShapeDtypeStruct((B, K, N, C), jnp.float32),
            jax.ShapeDtypeStruct((B, K, 1, C), jnp.float32),
        ),
    )(idx, patch, audio)


def kernel(top_k_index_sort, patch_feat, audio_feat):
    out_p, out_a = _gather_call(top_k_index_sort.astype(jnp.int32),
                                patch_feat, audio_feat.reshape(B, T, 1, C))
    return out_p, out_a.reshape(B, K, C)


# speedup vs baseline: 4.0292x; 1.0254x over previous
import jax
import jax.numpy as jnp
from jax.experimental import pallas as pl
from jax.experimental.pallas import tpu as pltpu

B, T, N, C, K = 8, 32, 256, 768, 10
R = B * K


def _body(idx_sm, patch_blk, audio, out_patch_blk, out_audio, asem):
    i = pl.program_id(0)

    @pl.when(i == 0)
    def _start_audio():
        for r in range(R):
            b, k = r // K, r % K
            t = idx_sm[b, 0, k]
            pltpu.make_async_copy(audio.at[b, t], out_audio.at[b, k],
                                  asem).start()

    out_patch_blk[...] = patch_blk[...]

    @pl.when(i == R - 1)
    def _drain_audio():
        for r in range(R):
            b, k = r // K, r % K
            t = idx_sm[b, 0, k]
            pltpu.make_async_copy(audio.at[b, t], out_audio.at[b, k],
                                  asem).wait()


@jax.jit
def _gather_call(idx, patch, audio):
    grid_spec = pltpu.PrefetchScalarGridSpec(
        num_scalar_prefetch=1,
        grid=(R,),
        in_specs=[
            pl.BlockSpec((1, 1, N, C),
                         lambda i, idx_ref: (i // K, idx_ref[i // K, 0, i % K], 0, 0)),
            pl.BlockSpec(memory_space=pl.ANY),
        ],
        out_specs=[
            pl.BlockSpec((1, 1, N, C), lambda i, idx_ref: (i // K, i % K, 0, 0)),
            pl.BlockSpec(memory_space=pl.ANY),
        ],
        scratch_shapes=[
            pltpu.SemaphoreType.DMA,
        ],
    )
    return pl.pallas_call(
        _body,
        grid_spec=grid_spec,
        out_shape=(
            jax.ShapeDtypeStruct((B, K, N, C), jnp.float32),
            jax.ShapeDtypeStruct((B, K, C), jnp.float32),
        ),
    )(idx, patch, audio)


def kernel(top_k_index_sort, patch_feat, audio_feat):
    return _gather_call(top_k_index_sort.astype(jnp.int32),
                        patch_feat, audio_feat)


# speedup vs baseline: 5.6860x; 1.4112x over previous
import jax
import jax.numpy as jnp
from jax.experimental import pallas as pl
from jax.experimental.pallas import tpu as pltpu

B, T, N, C, K = 8, 32, 256, 768, 10
R = B * K


def _body(idx_sm, patch_blk, patch_blk2, audio, out_patch_blk,
          out_audio, asem):
    i = pl.program_id(0)

    @pl.when(i == 0)
    def _start_audio():
        for r in range(R):
            b, k = r // K, r % K
            t = idx_sm[b, 0, k]
            pltpu.make_async_copy(audio.at[b, t], out_audio.at[b, k],
                                  asem).start()

    out_patch_blk[:, 0:1] = patch_blk[...]
    out_patch_blk[:, 1:2] = patch_blk2[...]

    @pl.when(i == R // 2 - 1)
    def _drain_audio():
        for r in range(R):
            b, k = r // K, r % K
            t = idx_sm[b, 0, k]
            pltpu.make_async_copy(audio.at[b, t], out_audio.at[b, k],
                                  asem).wait()


@jax.jit
def _gather_call(idx, patch, audio):
    grid_spec = pltpu.PrefetchScalarGridSpec(
        num_scalar_prefetch=1,
        grid=(R // 2,),
        in_specs=[
            pl.BlockSpec((1, 1, N, C),
                         lambda i, idx_ref: ((2 * i) // K, idx_ref[(2 * i) // K, 0, (2 * i) % K], 0, 0)),
            pl.BlockSpec((1, 1, N, C),
                         lambda i, idx_ref: ((2 * i + 1) // K, idx_ref[(2 * i + 1) // K, 0, (2 * i + 1) % K], 0, 0)),
            pl.BlockSpec(memory_space=pl.ANY),
        ],
        out_specs=[
            pl.BlockSpec((1, 2, N, C),
                         lambda i, idx_ref: (i // 5, i % 5, 0, 0)),
            pl.BlockSpec(memory_space=pl.ANY),
        ],
        scratch_shapes=[
            pltpu.SemaphoreType.DMA,
        ],
    )
    return pl.pallas_call(
        _body,
        grid_spec=grid_spec,
        out_shape=(
            jax.ShapeDtypeStruct((B, K, N, C), jnp.float32),
            jax.ShapeDtypeStruct((B, K, C), jnp.float32),
        ),
    )(idx, patch, patch, audio)


def kernel(top_k_index_sort, patch_feat, audio_feat):
    return _gather_call(top_k_index_sort.astype(jnp.int32),
                        patch_feat, audio_feat)


# speedup vs baseline: 6.7867x; 1.1936x over previous
import jax
import jax.numpy as jnp
from jax.experimental import pallas as pl
from jax.experimental.pallas import tpu as pltpu

B, T, N, C, K = 8, 32, 256, 768, 10
R = B * K


def _body(idx_sm, p0, p1, p2, p3, p4, audio, out_patch_blk,
          out_audio, asem):
    i = pl.program_id(0)

    @pl.when(i == 0)
    def _start_audio():
        for r in range(R):
            b, k = r // K, r % K
            t = idx_sm[b, 0, k]
            pltpu.make_async_copy(audio.at[b, t], out_audio.at[b, k],
                                  asem).start()

    for q, blk in enumerate((p0, p1, p2, p3, p4)):
        out_patch_blk[:, q:q + 1] = blk[...]

    @pl.when(i == R // 5 - 1)
    def _drain_audio():
        for r in range(R):
            b, k = r // K, r % K
            t = idx_sm[b, 0, k]
            pltpu.make_async_copy(audio.at[b, t], out_audio.at[b, k],
                                  asem).wait()


@jax.jit
def _gather_call(idx, patch, audio):
    grid_spec = pltpu.PrefetchScalarGridSpec(
        num_scalar_prefetch=1,
        grid=(R // 5,),
        in_specs=[
            pl.BlockSpec((1, 1, N, C),
                         (lambda q: (lambda i, idx_ref: (
                             (5 * i + q) // K,
                             idx_ref[(5 * i + q) // K, 0, (5 * i + q) % K],
                             0, 0)))(q))
            for q in range(5)
        ] + [
            pl.BlockSpec(memory_space=pl.ANY),
        ],
        out_specs=[
            pl.BlockSpec((1, 5, N, C),
                         lambda i, idx_ref: (i // 2, i % 2, 0, 0)),
            pl.BlockSpec(memory_space=pl.ANY),
        ],
        scratch_shapes=[
            pltpu.SemaphoreType.DMA,
        ],
    )
    return pl.pallas_call(
        _body,
        grid_spec=grid_spec,
        out_shape=(
            jax.ShapeDtypeStruct((B, K, N, C), jnp.float32),
            jax.ShapeDtypeStruct((B, K, C), jnp.float32),
        ),
    )(idx, patch, patch, patch, patch, patch, audio)


def kernel(top_k_index_sort, patch_feat, audio_feat):
    return _gather_call(top_k_index_sort.astype(jnp.int32),
                        patch_feat, audio_feat)


# speedup vs baseline: 6.9301x; 1.0211x over previous
import jax
import jax.numpy as jnp
from jax.experimental import pallas as pl
from jax.experimental.pallas import tpu as pltpu

B, T, N, C, K = 8, 32, 256, 768, 10
R = B * K


def _body(idx_sm, p0, p1, p2, p3, p4, p5, p6, p7, p8, p9, audio,
          out_patch_blk, out_audio, asem):
    i = pl.program_id(0)

    @pl.when(i == 0)
    def _start_audio():
        for r in range(R):
            b, k = r // K, r % K
            t = idx_sm[b, 0, k]
            pltpu.make_async_copy(audio.at[b, t], out_audio.at[b, k],
                                  asem).start()

    for q, blk in enumerate((p0, p1, p2, p3, p4, p5, p6, p7, p8, p9)):
        out_patch_blk[:, q:q + 1] = blk[...]

    @pl.when(i == R // 10 - 1)
    def _drain_audio():
        for r in range(R):
            b, k = r // K, r % K
            t = idx_sm[b, 0, k]
            pltpu.make_async_copy(audio.at[b, t], out_audio.at[b, k],
                                  asem).wait()


@jax.jit
def _gather_call(idx, patch, audio):
    grid_spec = pltpu.PrefetchScalarGridSpec(
        num_scalar_prefetch=1,
        grid=(R // 10,),
        in_specs=[
            pl.BlockSpec((1, 1, N, C),
                         (lambda q: (lambda i, idx_ref: (
                             (10 * i + q) // K,
                             idx_ref[(10 * i + q) // K, 0, (10 * i + q) % K],
                             0, 0)))(q))
            for q in range(10)
        ] + [
            pl.BlockSpec(memory_space=pl.ANY),
        ],
        out_specs=[
            pl.BlockSpec((1, 10, N, C),
                         lambda i, idx_ref: (i, 0, 0, 0)),
            pl.BlockSpec(memory_space=pl.ANY),
        ],
        scratch_shapes=[
            pltpu.SemaphoreType.DMA,
        ],
    )
    return pl.pallas_call(
        _body,
        grid_spec=grid_spec,
        out_shape=(
            jax.ShapeDtypeStruct((B, K, N, C), jnp.float32),
            jax.ShapeDtypeStruct((B, K, C), jnp.float32),
        ),
    )(idx, *([patch] * 10), audio)


def kernel(top_k_index_sort, patch_feat, audio_feat):
    return _gather_call(top_k_index_sort.astype(jnp.int32),
                        patch_feat, audio_feat)


# speedup vs baseline: 6.9807x; 1.0073x over previous
import jax
import jax.numpy as jnp
from jax.experimental import pallas as pl
from jax.experimental.pallas import tpu as pltpu

B, T, N, C, K = 8, 32, 256, 768, 10
R = B * K


def _body(idx_sm, p0, p1, p2, p3, p4, p5, p6, p7, p8, p9, audio,
          out_patch_blk, out_audio, asem):
    i = pl.program_id(0)

    @pl.when(i == 0)
    def _start_audio():
        for r in range(R):
            b, k = r // K, r % K
            t = idx_sm[b, 0, k]
            pltpu.make_async_copy(audio.at[b, t], out_audio.at[b, k],
                                  asem).start()

    for q, blk in enumerate((p0, p1, p2, p3, p4, p5, p6, p7, p8, p9)):
        out_patch_blk[:, q:q + 1] = blk[...]

    @pl.when(i == B - 1)
    def _drain_audio():
        for r in range(R):
            b, k = r // K, r % K
            t = idx_sm[b, 0, k]
            pltpu.make_async_copy(audio.at[b, t], out_audio.at[b, k],
                                  asem).wait()


@jax.jit
def _gather_call(idx, patch, audio):
    grid_spec = pltpu.PrefetchScalarGridSpec(
        num_scalar_prefetch=1,
        grid=(B,),
        in_specs=[
            pl.BlockSpec(
                (1, 1, N, C),
                (lambda q: lambda i, idx_ref: (i, idx_ref[i, 0, q], 0, 0))(q))
            for q in range(K)
        ] + [
            pl.BlockSpec(memory_space=pl.ANY),
        ],
        out_specs=[
            pl.BlockSpec((1, K, N, C), lambda i, idx_ref: (i, 0, 0, 0)),
            pl.BlockSpec(memory_space=pl.ANY),
        ],
        scratch_shapes=[
            pltpu.SemaphoreType.DMA,
        ],
    )
    return pl.pallas_call(
        _body,
        grid_spec=grid_spec,
        out_shape=(
            jax.ShapeDtypeStruct((B, K, N, C), jnp.float32),
            jax.ShapeDtypeStruct((B, K, C), jnp.float32),
        ),
    )(idx, *([patch] * K), audio)


def kernel(top_k_index_sort, patch_feat, audio_feat):
    return _gather_call(top_k_index_sort.astype(jnp.int32),
                        patch_feat, audio_feat)
